# Initial kernel scaffold; baseline (speedup 1.0000x reference)
import jax
import jax.numpy as jnp
from jax.experimental import pallas as pl


def kernel(input, edge_index, edge_values, W, b):
    raise NotImplementedError("write your pallas kernel here")


# speedup vs baseline: 5.3625x; 5.3625x over previous
import functools

import jax
import jax.numpy as jnp
from jax import lax
from jax.experimental import pallas as pl
from jax.experimental.pallas import tpu as pltpu
from jax.experimental.pallas import tpu_sc as plsc

N = 10000
E = 320000
D = 128

NC = 2
NS = 16
NW = NC * NS
CHUNK = 128
NCHUNKS = E // CHUNK
NRCHUNK = N // CHUNK
TAILR = N - NRCHUNK * CHUNK


def _mm_body(x_ref, wt_ref, b_ref, o_ref):
    o_ref[...] = (
        jnp.dot(x_ref[...], wt_ref[...], preferred_element_type=jnp.float32)
        + b_ref[...]
    )


def _support(x, wt, b2d):
    return pl.pallas_call(
        _mm_body,
        grid=(10,),
        in_specs=[
            pl.BlockSpec((N // 10, D), lambda i: (i, 0)),
            pl.BlockSpec((D, D), lambda i: (0, 0)),
            pl.BlockSpec((1, D), lambda i: (0, 0)),
        ],
        out_specs=pl.BlockSpec((N // 10, D), lambda i: (i, 0)),
        out_shape=jax.ShapeDtypeStruct((N, D), jnp.float32),
    )(x, wt, b2d)


def _add_body(a_ref, b_ref, o_ref):
    o_ref[...] = a_ref[...] + b_ref[...]


def _combine(p0, p1):
    return pl.pallas_call(
        _add_body,
        grid=(10,),
        in_specs=[
            pl.BlockSpec((N // 10, D), lambda i: (i, 0)),
            pl.BlockSpec((N // 10, D), lambda i: (i, 0)),
        ],
        out_specs=pl.BlockSpec((N // 10, D), lambda i: (i, 0)),
        out_shape=jax.ShapeDtypeStruct((N, D), jnp.float32),
    )(p0, p1)


def _bcast_lane(v, i):
    idx = jnp.full((16, 1), i, jnp.int32)
    dn = lax.GatherDimensionNumbers(
        offset_dims=(), collapsed_slice_dims=(0,), start_index_map=(0,)
    )
    return lax.gather(v, idx, dn, (1,),
                      mode=lax.GatherScatterMode.PROMISE_IN_BOUNDS)

_mesh = plsc.VectorSubcoreMesh(core_axis_name="c", subcore_axis_name="s")


@functools.partial(
    pl.kernel,
    out_type=[
        jax.ShapeDtypeStruct((N, D), jnp.float32),
        jax.ShapeDtypeStruct((N, D), jnp.float32),
    ],
    mesh=_mesh,
    scratch_types=[
        pltpu.VMEM((1, CHUNK), jnp.int32),
        pltpu.VMEM((1, CHUNK), jnp.int32),
        pltpu.VMEM((1, CHUNK), jnp.float32),
        pltpu.VMEM((CHUNK, D), jnp.float32),
        pltpu.VMEM_SHARED((N, D), jnp.float32),
        pltpu.SemaphoreType.DMA,
    ],
)
def _spmm(support_hbm, col_hbm, row_hbm, vals_hbm, p0_hbm, p1_hbm,
          cidx_v, ridx_v, vals_v, rows_v, acc_sh, sem):
    c = lax.axis_index("c")
    s = lax.axis_index("s")
    w = c * NS + s

    zero16 = jnp.zeros((16,), jnp.float32)

    def _zrow(r, carry):
        for f in range(D // 16):
            rows_v[r, pl.ds(f * 16, 16)] = zero16
        return carry

    lax.fori_loop(0, CHUNK, _zrow, 0)

    nrows_mine = NRCHUNK // NS + jnp.where(s < NRCHUNK - (NRCHUNK // NS) * NS,
                                           1, 0)

    def _zchunk(k, carry):
        off = pl.multiple_of((s + k * NS) * CHUNK, CHUNK)
        pltpu.sync_copy(rows_v, acc_sh.at[pl.ds(off, CHUNK)])
        return carry

    lax.fori_loop(0, nrows_mine, _zchunk, 0)

    @pl.when(s == 0)
    def _():
        pltpu.sync_copy(rows_v.at[pl.ds(0, TAILR)],
                        acc_sh.at[pl.ds(NRCHUNK * CHUNK, TAILR)])

    plsc.subcore_barrier()

    lane_iota_zero = jnp.zeros((16,), jnp.int32)

    def _chunk(k, carry):
        base = pl.multiple_of((w + k * NW) * CHUNK, CHUNK)
        pltpu.sync_copy(col_hbm.at[pl.ds(base, CHUNK)], cidx_v.at[0])
        pltpu.sync_copy(row_hbm.at[pl.ds(base, CHUNK)], ridx_v.at[0])
        pltpu.sync_copy(vals_hbm.at[pl.ds(base, CHUNK)], vals_v.at[0])
        pltpu.async_copy(support_hbm.at[cidx_v.at[0]], rows_v, sem).wait()

        def _scale(g, inner):
            vv = vals_v[0, pl.ds(g * 16, 16)]
            for i in range(16):
                bc = _bcast_lane(vv, i)
                r = g * 16 + i
                for f in range(D // 16):
                    rows_v[r, pl.ds(f * 16, 16)] = (
                        rows_v[r, pl.ds(f * 16, 16)] * bc
                    )
            return inner

        lax.fori_loop(0, CHUNK // 16, _scale, 0)

        pltpu.sync_copy(rows_v, acc_sh.at[ridx_v.at[0]], add=True)
        return carry

    nmine = jnp.where(w < NCHUNKS - (NCHUNKS // NW) * NW,
                      NCHUNKS // NW + 1, NCHUNKS // NW)
    lax.fori_loop(0, nmine, _chunk, 0)

    plsc.subcore_barrier()

    def _flush(dst_hbm):
        def _fchunk(k, carry):
            off = pl.multiple_of((s + k * NS) * CHUNK, CHUNK)
            pltpu.sync_copy(acc_sh.at[pl.ds(off, CHUNK)], rows_v)
            pltpu.sync_copy(rows_v, dst_hbm.at[pl.ds(off, CHUNK)])
            return carry

        lax.fori_loop(0, nrows_mine, _fchunk, 0)

        @pl.when(s == 0)
        def _():
            pltpu.sync_copy(acc_sh.at[pl.ds(NRCHUNK * CHUNK, TAILR)],
                            rows_v.at[pl.ds(0, TAILR)])
            pltpu.sync_copy(rows_v.at[pl.ds(0, TAILR)],
                            dst_hbm.at[pl.ds(NRCHUNK * CHUNK, TAILR)])

    @pl.when(c == 0)
    def _():
        _flush(p0_hbm)

    @pl.when(c == 1)
    def _():
        _flush(p1_hbm)


def kernel(input, edge_index, edge_values, W, b):
    ei = edge_index.astype(jnp.int32)
    row = ei[0]
    col = ei[1]
    support = _support(input, W.T, b.reshape(1, D))
    p0, p1 = _spmm(support, col, row, edge_values)
    return _combine(p0, p1)
